# baseline (speedup 1.0000x reference)
import functools

import jax
import jax.numpy as jnp
import numpy as np
from jax import lax
from jax.experimental import pallas as pl
from jax.experimental.pallas import tpu as pltpu
from jax.experimental.pallas import tpu_sc as plsc

_B, _S, _D, _SD = 4, 2048, 2048, 64
_INV_SQRT2 = 0.7071067811865476


def _proj_kernel(x_ref, w1_ref, b1_ref, w2_ref, b2_ref, xsrc_ref,
                 st_ref, x1_ref, sem):
    i = pl.program_id(0)

    @pl.when(i == 0)
    def _():
        pltpu.make_async_copy(xsrc_ref, x1_ref, sem).start()

    h = jnp.dot(x_ref[...], w1_ref[...], preferred_element_type=jnp.float32)
    h = h + b1_ref[...]
    h = 0.5 * h * (1.0 + lax.erf(h * _INV_SQRT2))
    st = jnp.dot(h, w2_ref[...], preferred_element_type=jnp.float32)
    st = st + b2_ref[...]
    nrm = jnp.sqrt(jnp.sum(st * st, axis=-1, keepdims=True))
    st_ref[...] = st / jnp.maximum(nrm, 1e-12)

    @pl.when(i == pl.num_programs(0) - 1)
    def _():
        pltpu.make_async_copy(xsrc_ref, x1_ref, sem).wait()


def _argmax_kernel(stb_ref, full_ref, xsrc_ref, idx_ref, x2_ref, sem, *, rows):
    b = pl.program_id(0)
    pid = b * pl.num_programs(1) + pl.program_id(1)

    @pl.when(pid == 0)
    def _():
        pltpu.make_async_copy(xsrc_ref, x2_ref, sem).start()

    a = stb_ref[0]
    f = full_ref[0]
    sim = lax.dot_general(a, f, (((1,), (1,)), ((), ())),
                          preferred_element_type=jnp.float32)
    m = jnp.max(sim, axis=1, keepdims=True)
    ii = lax.broadcasted_iota(jnp.int32, sim.shape, 1)
    first = jnp.min(jnp.where(sim >= m, ii, rows), axis=1)
    idx_ref[0, 0, 0, :] = first + b * rows

    @pl.when(pid == pl.num_programs(0) * pl.num_programs(1) - 1)
    def _():
        pltpu.make_async_copy(xsrc_ref, x2_ref, sem).wait()


def _compute_indices(x_flat, W1, b1, W2, b2):
    BS = _B * _S
    BR1 = 256
    st, x1 = pl.pallas_call(
        _proj_kernel,
        grid=(BS // BR1,),
        in_specs=[
            pl.BlockSpec((BR1, _D), lambda i: (i, 0)),
            pl.BlockSpec((_D, 2 * _SD), lambda i: (0, 0)),
            pl.BlockSpec((1, 2 * _SD), lambda i: (0, 0)),
            pl.BlockSpec((2 * _SD, _SD), lambda i: (0, 0)),
            pl.BlockSpec((1, _SD), lambda i: (0, 0)),
            pl.BlockSpec(memory_space=pl.ANY),
        ],
        out_specs=[
            pl.BlockSpec((BR1, _SD), lambda i: (i, 0)),
            pl.BlockSpec(memory_space=pl.ANY),
        ],
        out_shape=[
            jax.ShapeDtypeStruct((BS, _SD), jnp.float32),
            jax.ShapeDtypeStruct((BS, _D), jnp.float32),
        ],
        scratch_shapes=[pltpu.SemaphoreType.DMA],
    )(x_flat, W1, b1.reshape(1, -1), W2, b2.reshape(1, -1), x_flat)

    st3 = st.reshape(_B, _S, _SD)
    BR2 = 512
    NB = _S // BR2
    idx, x2 = pl.pallas_call(
        functools.partial(_argmax_kernel, rows=_S),
        grid=(_B, NB),
        in_specs=[
            pl.BlockSpec((1, BR2, _SD), lambda b, r: (b, r, 0)),
            pl.BlockSpec((1, _S, _SD), lambda b, r: (b, 0, 0)),
            pl.BlockSpec(memory_space=pl.ANY),
        ],
        out_specs=[
            pl.BlockSpec((1, 1, 1, BR2), lambda b, r: (b, r, 0, 0)),
            pl.BlockSpec(memory_space=pl.ANY),
        ],
        out_shape=[
            jax.ShapeDtypeStruct((_B, NB, 1, BR2), jnp.int32),
            jax.ShapeDtypeStruct((BS, _D), jnp.float32),
        ],
        scratch_shapes=[pltpu.SemaphoreType.DMA],
    )(st3, st3, x_flat)
    return idx.reshape(BS), x1, x2


def _make_sc_copy(BS, D):
    info = plsc.get_sparse_core_info()
    NC, NS = info.num_cores, info.num_subcores
    NW = NC * NS
    b_per_w = BS // NW
    mesh = plsc.VectorSubcoreMesh(core_axis_name="c", subcore_axis_name="s")

    @functools.partial(
        pl.kernel,
        mesh=mesh,
        out_type=jax.ShapeDtypeStruct((BS, D), jnp.float32),
        scratch_types=[pltpu.SemaphoreType.DMA],
    )
    def copyk(x_hbm, out_hbm, sem):
        wid = lax.axis_index("s") * NC + lax.axis_index("c")
        base = wid * b_per_w
        pltpu.async_copy(
            x_hbm.at[pl.ds(base, b_per_w)], out_hbm.at[pl.ds(base, b_per_w)], sem
        ).wait()

    return copyk


def _make_sc_gather(BS, D):
    info = plsc.get_sparse_core_info()
    NC, NS = info.num_cores, info.num_subcores
    NW = NC * NS
    b_per_w = BS // NW
    CH = 32
    n_chunks = b_per_w // CH
    mesh = plsc.VectorSubcoreMesh(core_axis_name="c", subcore_axis_name="s")

    @functools.partial(
        pl.kernel,
        mesh=mesh,
        out_type=jax.ShapeDtypeStruct((BS, D), jnp.float32),
        scratch_types=[
            pltpu.VMEM((CH,), jnp.int32),
            pltpu.VMEM((CH, D), jnp.float32),
            pltpu.SemaphoreType.DMA,
        ],
    )
    def gather(x_hbm, idx_hbm, out_hbm, idx_v, rows_v, sem):
        wid = lax.axis_index("s") * NC + lax.axis_index("c")
        base = wid * b_per_w
        for c in range(n_chunks):
            off = base + c * CH
            pltpu.sync_copy(idx_hbm.at[pl.ds(off, CH)], idx_v)
            pltpu.async_copy(x_hbm.at[idx_v], rows_v, sem).wait()
            pltpu.sync_copy(rows_v, out_hbm.at[pl.ds(off, CH)])

    return gather


def kernel(x, W1, b1, W2, b2):
    B, S, D = x.shape
    BS = B * S
    x_flat = x.reshape(BS, D)
    idx, x1, x2 = _compute_indices(x_flat, W1, b1, W2, b2)
    gather = _make_sc_gather(BS, D)
    q_flat = gather(x_flat, idx)
    return (q_flat.reshape(B, S, D), x1.reshape(B, S, D), x2.reshape(B, S, D))


# speedup vs baseline: 1.9318x; 1.9318x over previous
import functools

import jax
import jax.numpy as jnp
import numpy as np
from jax import lax
from jax.experimental import pallas as pl
from jax.experimental.pallas import tpu as pltpu
from jax.experimental.pallas import tpu_sc as plsc

_B, _S, _D, _SD = 4, 2048, 2048, 64
_INV_SQRT2 = 0.7071067811865476


def _proj_kernel(x_ref, w1_ref, b1_ref, w2_ref, b2_ref, st_ref):
    h = jnp.dot(x_ref[...], w1_ref[...], preferred_element_type=jnp.float32)
    h = h + b1_ref[...]
    h = 0.5 * h * (1.0 + lax.erf(h * _INV_SQRT2))
    st = jnp.dot(h, w2_ref[...], preferred_element_type=jnp.float32)
    st = st + b2_ref[...]
    nrm = jnp.sqrt(jnp.sum(st * st, axis=-1, keepdims=True))
    st_ref[...] = st / jnp.maximum(nrm, 1e-12)


def _argmax_kernel(stb_ref, full_ref, idx_ref, *, rows):
    b = pl.program_id(0)
    a = stb_ref[0]
    f = full_ref[0]
    sim = lax.dot_general(a, f, (((1,), (1,)), ((), ())),
                          preferred_element_type=jnp.float32)
    m = jnp.max(sim, axis=1, keepdims=True)
    ii = lax.broadcasted_iota(jnp.int32, sim.shape, 1)
    first = jnp.min(jnp.where(sim >= m, ii, rows), axis=1)
    idx_ref[0, 0, 0, :] = first + b * rows


def _compute_indices(x_flat, W1, b1, W2, b2):
    BS = _B * _S
    BR1 = 256
    st = pl.pallas_call(
        _proj_kernel,
        grid=(BS // BR1,),
        in_specs=[
            pl.BlockSpec((BR1, _D), lambda i: (i, 0)),
            pl.BlockSpec((_D, 2 * _SD), lambda i: (0, 0)),
            pl.BlockSpec((1, 2 * _SD), lambda i: (0, 0)),
            pl.BlockSpec((2 * _SD, _SD), lambda i: (0, 0)),
            pl.BlockSpec((1, _SD), lambda i: (0, 0)),
        ],
        out_specs=pl.BlockSpec((BR1, _SD), lambda i: (i, 0)),
        out_shape=jax.ShapeDtypeStruct((BS, _SD), jnp.float32),
    )(x_flat, W1, b1.reshape(1, -1), W2, b2.reshape(1, -1))

    st3 = st.reshape(_B, _S, _SD)
    BR2 = 512
    NB = _S // BR2
    idx = pl.pallas_call(
        functools.partial(_argmax_kernel, rows=_S),
        grid=(_B, NB),
        in_specs=[
            pl.BlockSpec((1, BR2, _SD), lambda b, r: (b, r, 0)),
            pl.BlockSpec((1, _S, _SD), lambda b, r: (b, 0, 0)),
        ],
        out_specs=pl.BlockSpec((1, 1, 1, BR2), lambda b, r: (b, r, 0, 0)),
        out_shape=jax.ShapeDtypeStruct((_B, NB, 1, BR2), jnp.int32),
    )(st3, st3)
    return idx.reshape(BS)


def _copy1_kernel(idx_ref, x_ref, o_ref):
    del idx_ref
    o_ref[...] = x_ref[...]


def _copy1(x_flat, idx):
    BS, D = x_flat.shape
    BR = 512
    o = pl.pallas_call(
        _copy1_kernel,
        grid=(BS // BR,),
        in_specs=[
            pl.BlockSpec(memory_space=pl.ANY),
            pl.BlockSpec((BR, D), lambda i: (i, 0)),
        ],
        out_specs=pl.BlockSpec((BR, D), lambda i: (i, 0)),
        out_shape=jax.ShapeDtypeStruct((BS, D), jnp.float32),
    )(idx, x_flat)
    return o


def _make_sc_copy(BS, D):
    info = plsc.get_sparse_core_info()
    NC, NS = info.num_cores, info.num_subcores
    NW = NC * NS
    b_per_w = BS // NW
    mesh = plsc.VectorSubcoreMesh(core_axis_name="c", subcore_axis_name="s")

    @functools.partial(
        pl.kernel,
        mesh=mesh,
        out_type=jax.ShapeDtypeStruct((BS, D), jnp.float32),
        scratch_types=[pltpu.SemaphoreType.DMA],
    )
    def copyk(x_hbm, out_hbm, sem):
        wid = lax.axis_index("s") * NC + lax.axis_index("c")
        base = wid * b_per_w
        pltpu.async_copy(
            x_hbm.at[pl.ds(base, b_per_w)], out_hbm.at[pl.ds(base, b_per_w)], sem
        ).wait()

    return copyk


def _make_sc_gather(BS, D):
    info = plsc.get_sparse_core_info()
    NC, NS = info.num_cores, info.num_subcores
    NW = NC * NS
    b_per_w = BS // NW
    CH = 32
    n_chunks = b_per_w // CH
    mesh = plsc.VectorSubcoreMesh(core_axis_name="c", subcore_axis_name="s")

    @functools.partial(
        pl.kernel,
        mesh=mesh,
        out_type=jax.ShapeDtypeStruct((BS, D), jnp.float32),
        scratch_types=[
            pltpu.VMEM((CH,), jnp.int32),
            pltpu.VMEM((CH, D), jnp.float32),
            pltpu.SemaphoreType.DMA,
        ],
    )
    def gather(x_hbm, idx_hbm, x1_hbm, out_hbm, idx_v, rows_v, sem):
        del x1_hbm
        wid = lax.axis_index("s") * NC + lax.axis_index("c")
        base = wid * b_per_w
        for c in range(n_chunks):
            off = base + c * CH
            pltpu.sync_copy(idx_hbm.at[pl.ds(off, CH)], idx_v)
            pltpu.async_copy(x_hbm.at[idx_v], rows_v, sem).wait()
            pltpu.sync_copy(rows_v, out_hbm.at[pl.ds(off, CH)])

    return gather


def kernel(x, W1, b1, W2, b2):
    B, S, D = x.shape
    BS = B * S
    x_flat = x.reshape(BS, D)
    x1_flat = _make_sc_copy(BS, D)(x_flat)
    idx = _compute_indices(x_flat, W1, b1, W2, b2)
    q_flat = _make_sc_gather(BS, D)(x_flat, idx, x1_flat)
    x2_flat = _copy1(x_flat, idx)
    return (q_flat.reshape(B, S, D), x1_flat.reshape(B, S, D),
            x2_flat.reshape(B, S, D))


# speedup vs baseline: 19.3943x; 10.0394x over previous
import functools

import jax
import jax.numpy as jnp
import numpy as np
from jax import lax
from jax.experimental import pallas as pl
from jax.experimental.pallas import tpu as pltpu
from jax.experimental.pallas import tpu_sc as plsc

_B, _S, _D, _SD = 4, 2048, 2048, 64
_INV_SQRT2 = 0.7071067811865476


def _proj_kernel(x_ref, w1_ref, b1_ref, w2_ref, b2_ref, st_ref):
    h = jnp.dot(x_ref[...], w1_ref[...], preferred_element_type=jnp.float32)
    h = h + b1_ref[...]
    h = 0.5 * h * (1.0 + lax.erf(h * _INV_SQRT2))
    st = jnp.dot(h, w2_ref[...], preferred_element_type=jnp.float32)
    st = st + b2_ref[...]
    nrm = jnp.sqrt(jnp.sum(st * st, axis=-1, keepdims=True))
    st_ref[...] = st / jnp.maximum(nrm, 1e-12)


def _argmax_kernel(stb_ref, full_ref, idx_ref, *, rows):
    b = pl.program_id(0)
    a = stb_ref[0]
    f = full_ref[0]
    sim = lax.dot_general(a, f, (((1,), (1,)), ((), ())),
                          preferred_element_type=jnp.float32)
    m = jnp.max(sim, axis=1, keepdims=True)
    ii = lax.broadcasted_iota(jnp.int32, sim.shape, 1)
    first = jnp.min(jnp.where(sim >= m, ii, rows), axis=1)
    idx_ref[0, 0, 0, :] = first + b * rows


def _compute_indices(x_flat, W1, b1, W2, b2):
    BS = _B * _S
    BR1 = 256
    st = pl.pallas_call(
        _proj_kernel,
        grid=(BS // BR1,),
        in_specs=[
            pl.BlockSpec((BR1, _D), lambda i: (i, 0)),
            pl.BlockSpec((_D, 2 * _SD), lambda i: (0, 0)),
            pl.BlockSpec((1, 2 * _SD), lambda i: (0, 0)),
            pl.BlockSpec((2 * _SD, _SD), lambda i: (0, 0)),
            pl.BlockSpec((1, _SD), lambda i: (0, 0)),
        ],
        out_specs=pl.BlockSpec((BR1, _SD), lambda i: (i, 0)),
        out_shape=jax.ShapeDtypeStruct((BS, _SD), jnp.float32),
    )(x_flat, W1, b1.reshape(1, -1), W2, b2.reshape(1, -1))

    st3 = st.reshape(_B, _S, _SD)
    BR2 = 512
    NB = _S // BR2
    idx = pl.pallas_call(
        functools.partial(_argmax_kernel, rows=_S),
        grid=(_B, NB),
        in_specs=[
            pl.BlockSpec((1, BR2, _SD), lambda b, r: (b, r, 0)),
            pl.BlockSpec((1, _S, _SD), lambda b, r: (b, 0, 0)),
        ],
        out_specs=pl.BlockSpec((1, 1, 1, BR2), lambda b, r: (b, r, 0, 0)),
        out_shape=jax.ShapeDtypeStruct((_B, NB, 1, BR2), jnp.int32),
    )(st3, st3)
    return idx.reshape(BS)


def _copy1_kernel(idx_ref, x_ref, o_ref):
    del idx_ref
    o_ref[...] = x_ref[...]


def _copy1(x_flat, idx):
    BS, D = x_flat.shape
    BR = 512
    o = pl.pallas_call(
        _copy1_kernel,
        grid=(BS // BR,),
        in_specs=[
            pl.BlockSpec(memory_space=pl.ANY),
            pl.BlockSpec((BR, D), lambda i: (i, 0)),
        ],
        out_specs=pl.BlockSpec((BR, D), lambda i: (i, 0)),
        out_shape=jax.ShapeDtypeStruct((BS, D), jnp.float32),
    )(idx, x_flat)
    return o


def _make_sc_copy(BS, D):
    info = plsc.get_sparse_core_info()
    NC, NS = info.num_cores, info.num_subcores
    NW = NC * NS
    b_per_w = BS // NW
    mesh = plsc.VectorSubcoreMesh(core_axis_name="c", subcore_axis_name="s")

    CH = 32
    n_chunks = b_per_w // CH

    @functools.partial(
        pl.kernel,
        mesh=mesh,
        out_type=jax.ShapeDtypeStruct((BS, D), jnp.float32),
        scratch_types=[
            pltpu.VMEM((CH, D), jnp.float32),
            pltpu.SemaphoreType.DMA,
        ],
    )
    def copyk(x_hbm, out_hbm, rows_v, sem):
        wid = lax.axis_index("s") * NC + lax.axis_index("c")
        base = wid * b_per_w
        for c in range(n_chunks):
            off = base + c * CH
            pltpu.async_copy(x_hbm.at[pl.ds(off, CH)], rows_v, sem).wait()
            pltpu.sync_copy(rows_v, out_hbm.at[pl.ds(off, CH)])

    return copyk


def _make_sc_gather(BS, D):
    info = plsc.get_sparse_core_info()
    NC, NS = info.num_cores, info.num_subcores
    NW = NC * NS
    b_per_w = BS // NW
    CH = 32
    n_chunks = b_per_w // CH
    mesh = plsc.VectorSubcoreMesh(core_axis_name="c", subcore_axis_name="s")

    @functools.partial(
        pl.kernel,
        mesh=mesh,
        out_type=jax.ShapeDtypeStruct((BS, D), jnp.float32),
        scratch_types=[
            pltpu.VMEM((CH,), jnp.int32),
            pltpu.VMEM((CH, D), jnp.float32),
            pltpu.SemaphoreType.DMA,
        ],
    )
    def gather(x_hbm, idx_hbm, x1_hbm, out_hbm, idx_v, rows_v, sem):
        del x1_hbm
        wid = lax.axis_index("s") * NC + lax.axis_index("c")
        base = wid * b_per_w
        for c in range(n_chunks):
            off = base + c * CH
            pltpu.sync_copy(idx_hbm.at[pl.ds(off, CH)], idx_v)
            pltpu.async_copy(x_hbm.at[idx_v], rows_v, sem).wait()
            pltpu.sync_copy(rows_v, out_hbm.at[pl.ds(off, CH)])

    return gather


def kernel(x, W1, b1, W2, b2):
    B, S, D = x.shape
    BS = B * S
    x_flat = x.reshape(BS, D)
    x1_flat = _make_sc_copy(BS, D)(x_flat)
    idx = _compute_indices(x_flat, W1, b1, W2, b2)
    q_flat = _make_sc_gather(BS, D)(x_flat, idx, x1_flat)
    x2_flat = _copy1(x_flat, idx)
    return (q_flat.reshape(B, S, D), x1_flat.reshape(B, S, D),
            x2_flat.reshape(B, S, D))


# speedup vs baseline: 20.5509x; 1.0596x over previous
import functools

import jax
import jax.numpy as jnp
import numpy as np
from jax import lax
from jax.experimental import pallas as pl
from jax.experimental.pallas import tpu as pltpu
from jax.experimental.pallas import tpu_sc as plsc

_B, _S, _D, _SD = 4, 2048, 2048, 64
_INV_SQRT2 = 0.7071067811865476


def _proj_kernel(x_ref, w1_ref, b1_ref, w2_ref, b2_ref, st_ref):
    xb = x_ref[...].astype(jnp.bfloat16)
    h = jnp.dot(xb, w1_ref[...], preferred_element_type=jnp.float32)
    h = h + b1_ref[...]
    h = 0.5 * h * (1.0 + lax.erf(h * _INV_SQRT2))
    st = jnp.dot(h.astype(jnp.bfloat16), w2_ref[...],
                 preferred_element_type=jnp.float32)
    st = st + b2_ref[...]
    nrm = jnp.sqrt(jnp.sum(st * st, axis=-1, keepdims=True))
    st_ref[...] = (st / jnp.maximum(nrm, 1e-12)).astype(jnp.bfloat16)


def _argmax_kernel(stb_ref, full_ref, idx_ref, *, rows):
    b = pl.program_id(0)
    a = stb_ref[0]
    f = full_ref[0]
    sim = lax.dot_general(a, f, (((1,), (1,)), ((), ())),
                          preferred_element_type=jnp.float32)
    m = jnp.max(sim, axis=1, keepdims=True)
    ii = lax.broadcasted_iota(jnp.int32, sim.shape, 1)
    first = jnp.min(jnp.where(sim >= m, ii, rows), axis=1)
    idx_ref[0, 0, 0, :] = first + b * rows


def _compute_indices(x_flat, W1, b1, W2, b2):
    BS = _B * _S
    BR1 = 256
    st = pl.pallas_call(
        _proj_kernel,
        grid=(BS // BR1,),
        in_specs=[
            pl.BlockSpec((BR1, _D), lambda i: (i, 0)),
            pl.BlockSpec((_D, 2 * _SD), lambda i: (0, 0)),
            pl.BlockSpec((1, 2 * _SD), lambda i: (0, 0)),
            pl.BlockSpec((2 * _SD, _SD), lambda i: (0, 0)),
            pl.BlockSpec((1, _SD), lambda i: (0, 0)),
        ],
        out_specs=pl.BlockSpec((BR1, _SD), lambda i: (i, 0)),
        out_shape=jax.ShapeDtypeStruct((BS, _SD), jnp.bfloat16),
    )(x_flat, W1.astype(jnp.bfloat16), b1.reshape(1, -1),
      W2.astype(jnp.bfloat16), b2.reshape(1, -1))

    st3 = st.reshape(_B, _S, _SD)
    BR2 = 512
    NB = _S // BR2
    idx = pl.pallas_call(
        functools.partial(_argmax_kernel, rows=_S),
        grid=(_B, NB),
        in_specs=[
            pl.BlockSpec((1, BR2, _SD), lambda b, r: (b, r, 0)),
            pl.BlockSpec((1, _S, _SD), lambda b, r: (b, 0, 0)),
        ],
        out_specs=pl.BlockSpec((1, 1, 1, BR2), lambda b, r: (b, r, 0, 0)),
        out_shape=jax.ShapeDtypeStruct((_B, NB, 1, BR2), jnp.int32),
    )(st3, st3)
    return idx.reshape(BS)


def _copy2_kernel(idx_ref, x_ref, o1_ref, o2_ref):
    del idx_ref
    v = x_ref[...]
    o1_ref[...] = v
    o2_ref[...] = v


def _copy2(x_flat, idx):
    BS, D = x_flat.shape
    BR = 512
    o1, o2 = pl.pallas_call(
        _copy2_kernel,
        grid=(BS // BR,),
        in_specs=[
            pl.BlockSpec(memory_space=pl.ANY),
            pl.BlockSpec((BR, D), lambda i: (i, 0)),
        ],
        out_specs=[
            pl.BlockSpec((BR, D), lambda i: (i, 0)),
            pl.BlockSpec((BR, D), lambda i: (i, 0)),
        ],
        out_shape=[
            jax.ShapeDtypeStruct((BS, D), jnp.float32),
            jax.ShapeDtypeStruct((BS, D), jnp.float32),
        ],
    )(idx, x_flat)
    return o1, o2


def _make_sc_copy(BS, D):
    info = plsc.get_sparse_core_info()
    NC, NS = info.num_cores, info.num_subcores
    NW = NC * NS
    b_per_w = BS // NW
    mesh = plsc.VectorSubcoreMesh(core_axis_name="c", subcore_axis_name="s")

    CH = 32
    n_chunks = b_per_w // CH

    @functools.partial(
        pl.kernel,
        mesh=mesh,
        out_type=jax.ShapeDtypeStruct((BS, D), jnp.float32),
        scratch_types=[
            pltpu.VMEM((CH, D), jnp.float32),
            pltpu.SemaphoreType.DMA,
        ],
    )
    def copyk(x_hbm, out_hbm, rows_v, sem):
        wid = lax.axis_index("s") * NC + lax.axis_index("c")
        base = wid * b_per_w
        for c in range(n_chunks):
            off = base + c * CH
            pltpu.async_copy(x_hbm.at[pl.ds(off, CH)], rows_v, sem).wait()
            pltpu.sync_copy(rows_v, out_hbm.at[pl.ds(off, CH)])

    return copyk


def _make_sc_gather(BS, D):
    info = plsc.get_sparse_core_info()
    NC, NS = info.num_cores, info.num_subcores
    NW = NC * NS
    b_per_w = BS // NW
    CH = 32
    n_chunks = b_per_w // CH
    mesh = plsc.VectorSubcoreMesh(core_axis_name="c", subcore_axis_name="s")

    @functools.partial(
        pl.kernel,
        mesh=mesh,
        out_type=jax.ShapeDtypeStruct((BS, D), jnp.float32),
        scratch_types=[
            pltpu.VMEM((CH,), jnp.int32),
            pltpu.VMEM((CH, D), jnp.float32),
            pltpu.SemaphoreType.DMA,
        ],
    )
    def gather(x_hbm, idx_hbm, out_hbm, idx_v, rows_v, sem):
        wid = lax.axis_index("s") * NC + lax.axis_index("c")
        base = wid * b_per_w
        for c in range(n_chunks):
            off = base + c * CH
            pltpu.sync_copy(idx_hbm.at[pl.ds(off, CH)], idx_v)
            pltpu.async_copy(x_hbm.at[idx_v], rows_v, sem).wait()
            pltpu.sync_copy(rows_v, out_hbm.at[pl.ds(off, CH)])

    return gather


def kernel(x, W1, b1, W2, b2):
    B, S, D = x.shape
    BS = B * S
    x_flat = x.reshape(BS, D)
    idx = _compute_indices(x_flat, W1, b1, W2, b2)
    q_flat = _make_sc_gather(BS, D)(x_flat, idx)
    x1_flat, x2_flat = _copy2(x_flat, idx)
    return (q_flat.reshape(B, S, D), x1_flat.reshape(B, S, D),
            x2_flat.reshape(B, S, D))


# speedup vs baseline: 23.4534x; 1.1412x over previous
import functools

import jax
import jax.numpy as jnp
import numpy as np
from jax import lax
from jax.experimental import pallas as pl
from jax.experimental.pallas import tpu as pltpu
from jax.experimental.pallas import tpu_sc as plsc

_B, _S, _D, _SD = 4, 2048, 2048, 64
_INV_SQRT2 = 0.7071067811865476


def _proj_kernel(x_ref, w1_ref, b1_ref, w2_ref, b2_ref, st_ref, x1_ref):
    xv = x_ref[...]
    h = jnp.dot(xv.astype(jnp.bfloat16), w1_ref[...],
                preferred_element_type=jnp.float32)
    h = h + b1_ref[...]
    h = 0.5 * h * (1.0 + lax.erf(h * _INV_SQRT2))
    st = jnp.dot(h.astype(jnp.bfloat16), w2_ref[...],
                 preferred_element_type=jnp.float32)
    st = st + b2_ref[...]
    nrm = jnp.sqrt(jnp.sum(st * st, axis=-1, keepdims=True))
    st_ref[...] = (st / jnp.maximum(nrm, 1e-12)).astype(jnp.bfloat16)
    x1_ref[...] = xv


def _argmax_kernel(stb_ref, full_ref, x_ref, idx_ref, x2_ref, *, rows):
    b = pl.program_id(0)
    a = stb_ref[0]
    f = full_ref[0]
    sim = lax.dot_general(a, f, (((1,), (1,)), ((), ())),
                          preferred_element_type=jnp.float32)
    m = jnp.max(sim, axis=1, keepdims=True)
    ii = lax.broadcasted_iota(jnp.int32, sim.shape, 1)
    first = jnp.min(jnp.where(sim >= m, ii, rows), axis=1)
    idx_ref[0, 0, 0, :] = first + b * rows
    x2_ref[...] = x_ref[...]


def _compute_indices(x_flat, W1, b1, W2, b2):
    BS = _B * _S
    BR1 = 256
    st = pl.pallas_call(
        _proj_kernel,
        grid=(BS // BR1,),
        in_specs=[
            pl.BlockSpec((BR1, _D), lambda i: (i, 0)),
            pl.BlockSpec((_D, 2 * _SD), lambda i: (0, 0)),
            pl.BlockSpec((1, 2 * _SD), lambda i: (0, 0)),
            pl.BlockSpec((2 * _SD, _SD), lambda i: (0, 0)),
            pl.BlockSpec((1, _SD), lambda i: (0, 0)),
        ],
        out_specs=[
            pl.BlockSpec((BR1, _SD), lambda i: (i, 0)),
            pl.BlockSpec((BR1, _D), lambda i: (i, 0)),
        ],
        out_shape=[
            jax.ShapeDtypeStruct((BS, _SD), jnp.bfloat16),
            jax.ShapeDtypeStruct((BS, _D), jnp.float32),
        ],
    )(x_flat, W1.astype(jnp.bfloat16), b1.reshape(1, -1),
      W2.astype(jnp.bfloat16), b2.reshape(1, -1))
    st, x1 = st

    st3 = st.reshape(_B, _S, _SD)
    x3 = x_flat.reshape(_B, _S, _D)
    BR2 = 512
    NB = _S // BR2
    idx, x2 = pl.pallas_call(
        functools.partial(_argmax_kernel, rows=_S),
        grid=(_B, NB),
        in_specs=[
            pl.BlockSpec((1, BR2, _SD), lambda b, r: (b, r, 0)),
            pl.BlockSpec((1, _S, _SD), lambda b, r: (b, 0, 0)),
            pl.BlockSpec((1, BR2, _D), lambda b, r: (b, r, 0)),
        ],
        out_specs=[
            pl.BlockSpec((1, 1, 1, BR2), lambda b, r: (b, r, 0, 0)),
            pl.BlockSpec((1, BR2, _D), lambda b, r: (b, r, 0)),
        ],
        out_shape=[
            jax.ShapeDtypeStruct((_B, NB, 1, BR2), jnp.int32),
            jax.ShapeDtypeStruct((_B, _S, _D), jnp.float32),
        ],
    )(st3, st3, x3)
    return idx.reshape(BS), x1, x2.reshape(BS, _D)


def _copy2_kernel(idx_ref, x_ref, o1_ref, o2_ref):
    del idx_ref
    v = x_ref[...]
    o1_ref[...] = v
    o2_ref[...] = v


def _copy2(x_flat, idx):
    BS, D = x_flat.shape
    BR = 512
    o1, o2 = pl.pallas_call(
        _copy2_kernel,
        grid=(BS // BR,),
        in_specs=[
            pl.BlockSpec(memory_space=pl.ANY),
            pl.BlockSpec((BR, D), lambda i: (i, 0)),
        ],
        out_specs=[
            pl.BlockSpec((BR, D), lambda i: (i, 0)),
            pl.BlockSpec((BR, D), lambda i: (i, 0)),
        ],
        out_shape=[
            jax.ShapeDtypeStruct((BS, D), jnp.float32),
            jax.ShapeDtypeStruct((BS, D), jnp.float32),
        ],
    )(idx, x_flat)
    return o1, o2


def _make_sc_copy(BS, D):
    info = plsc.get_sparse_core_info()
    NC, NS = info.num_cores, info.num_subcores
    NW = NC * NS
    b_per_w = BS // NW
    mesh = plsc.VectorSubcoreMesh(core_axis_name="c", subcore_axis_name="s")

    CH = 32
    n_chunks = b_per_w // CH

    @functools.partial(
        pl.kernel,
        mesh=mesh,
        out_type=jax.ShapeDtypeStruct((BS, D), jnp.float32),
        scratch_types=[
            pltpu.VMEM((CH, D), jnp.float32),
            pltpu.SemaphoreType.DMA,
        ],
    )
    def copyk(x_hbm, out_hbm, rows_v, sem):
        wid = lax.axis_index("s") * NC + lax.axis_index("c")
        base = wid * b_per_w
        for c in range(n_chunks):
            off = base + c * CH
            pltpu.async_copy(x_hbm.at[pl.ds(off, CH)], rows_v, sem).wait()
            pltpu.sync_copy(rows_v, out_hbm.at[pl.ds(off, CH)])

    return copyk


def _make_sc_gather(BS, D):
    info = plsc.get_sparse_core_info()
    NC, NS = info.num_cores, info.num_subcores
    NW = NC * NS
    b_per_w = BS // NW
    CH = 32
    n_chunks = b_per_w // CH
    mesh = plsc.VectorSubcoreMesh(core_axis_name="c", subcore_axis_name="s")

    @functools.partial(
        pl.kernel,
        mesh=mesh,
        out_type=jax.ShapeDtypeStruct((BS, D), jnp.float32),
        scratch_types=[
            pltpu.VMEM((CH,), jnp.int32),
            pltpu.VMEM((CH, D), jnp.float32),
            pltpu.SemaphoreType.DMA,
        ],
    )
    def gather(x_hbm, idx_hbm, out_hbm, idx_v, rows_v, sem):
        wid = lax.axis_index("s") * NC + lax.axis_index("c")
        base = wid * b_per_w
        for c in range(n_chunks):
            off = base + c * CH
            pltpu.sync_copy(idx_hbm.at[pl.ds(off, CH)], idx_v)
            pltpu.async_copy(x_hbm.at[idx_v], rows_v, sem).wait()
            pltpu.sync_copy(rows_v, out_hbm.at[pl.ds(off, CH)])

    return gather


def kernel(x, W1, b1, W2, b2):
    B, S, D = x.shape
    BS = B * S
    x_flat = x.reshape(BS, D)
    idx, x1_flat, x2_flat = _compute_indices(x_flat, W1, b1, W2, b2)
    q_flat = _make_sc_gather(BS, D)(x_flat, idx)
    return (q_flat.reshape(B, S, D), x1_flat.reshape(B, S, D),
            x2_flat.reshape(B, S, D))
